# baseline (speedup 1.0000x reference)
import functools

import jax
import jax.numpy as jnp
from jax import lax
from jax.experimental import pallas as pl
from jax.experimental.pallas import tpu as pltpu
from jax.experimental.pallas import tpu_sc as plsc

N = 1024
E = 16384
B = 2
NC, NS = 2, 16
EPW = E // NS
WPW = (N * N) // NS
ZB = 4096

_CONVS = [
    (("conv_node",), 3, 32, 5, "bn_node"),
    (("conv_0",), 30, 32, 5, "bn_0"),
    (("block1", "lc1"), 32, 64, 5, ("block1", "lb1")),
    (("block1", "lc2"), 64, 64, 5, ("block1", "lb2")),
    (("block1", "sc"), 32, 64, 1, ("block1", "sb")),
    (("block2", "lc1"), 64, 128, 5, ("block2", "lb1")),
    (("block2", "lc2"), 128, 128, 5, ("block2", "lb2")),
    (("block2", "sc"), 64, 128, 1, ("block2", "sb")),
    (("block3", "lc1"), 128, 256, 5, ("block3", "lb1")),
    (("block3", "lc2"), 256, 256, 5, ("block3", "lb2")),
    (("block3", "sc"), 128, 256, 1, ("block3", "sb")),
    (("block4", "lc1"), 256, 512, 5, ("block4", "lb1")),
    (("block4", "lc2"), 512, 512, 5, ("block4", "lb2")),
    (("block4", "sc"), 256, 512, 1, ("block4", "sb")),
]
_S0 = []
_acc = 0
for _i, _c in enumerate(_CONVS):
    if _i == 2:
        _acc += 1
    _S0.append(_acc)
    _acc += _c[3]
ONES_ROW = 10
NSLOT = _acc
SPLITS = ((0, 11), (11, 22), (33, 22))
ZERO_ROW = NSLOT
NGAUSS = NSLOT + 1


def _get(params, path):
    v = params
    for k in path:
        v = v[k]
    return v


def _sc_body(dst_hbm, src_hbm, gauss_hbm, a_out,
             dv, sv, idx2d, gnew, gold, delta3, zbuf, a_sp, sem,
             *, lo, nslots):
    c = lax.axis_index("c")
    s = lax.axis_index("s")
    base_e = s * EPW

    pltpu.sync_copy(dst_hbm.at[pl.ds(base_e, EPW)], dv)
    pltpu.sync_copy(src_hbm.at[pl.ds(base_e, EPW)], sv)

    for j in range(8):
        def mk(i, _, j=j):
            d = dv[pl.ds(j * 128 + i * 16, 16)]
            r = sv[pl.ds(j * 128 + i * 16, 16)]
            idx2d[j, pl.ds(i * 16, 16)] = (
                lax.shift_left(lax.shift_right_logical(r, 7), 17)
                + lax.shift_left(d, 7) + lax.bitwise_and(r, 127))
            return 0
        lax.fori_loop(0, 8, mk, 0)

    def z(i, _):
        zbuf[pl.ds(i * 16, 16)] = jnp.zeros((16,), jnp.float32)
        return 0
    lax.fori_loop(0, ZB // 16, z, 0)
    for q in range(WPW // ZB):
        pltpu.sync_copy(zbuf, a_sp.at[pl.ds(s * WPW + q * ZB, ZB)])

    n_iter = (nslots + 1) // 2
    hi = lo + nslots

    def mkdelta(i, _):
        slot = lo + 2 * i + c
        prev = jnp.where(i == 0, ZERO_ROW, slot - 2)
        sl = jnp.where(slot < hi, slot, ZERO_ROW)
        c1 = pltpu.async_copy(gauss_hbm.at[sl, pl.ds(base_e, EPW)], gnew, sem)
        c2 = pltpu.async_copy(gauss_hbm.at[prev, pl.ds(base_e, EPW)], gold,
                              sem)
        c1.wait()
        c2.wait()
        for j in range(8):
            def db(i2, _, j=j):
                delta3[i, j, pl.ds(i2 * 16, 16)] = (
                    gnew[pl.ds(j * 128 + i2 * 16, 16)]
                    - gold[pl.ds(j * 128 + i2 * 16, 16)])
                return 0
            lax.fori_loop(0, 8, db, 0)
        return 0

    lax.fori_loop(0, n_iter, mkdelta, 0)
    plsc.subcore_barrier()

    def slot_body(i, _):
        slot = lo + 2 * i + c

        @pl.when(slot < hi)
        def _():
            cps = [pltpu.async_copy(delta3.at[i, j], a_sp.at[idx2d.at[j]],
                                    sem, add=True) for j in range(8)]
            for cp in cps:
                cp.wait()
        plsc.subcore_barrier()

        @pl.when(slot < hi)
        def _():
            pltpu.sync_copy(
                a_sp.at[pl.ds(s * WPW, WPW)],
                a_out.at[pl.ds((slot - lo) * (N * N) + s * WPW, WPW)])
        plsc.subcore_barrier()
        return 0

    lax.fori_loop(0, n_iter, slot_body, 0)


@functools.cache
def _sc_build_fn(lo, nslots):
    return functools.partial(
        pl.kernel,
        out_type=jax.ShapeDtypeStruct((nslots * N * N,), jnp.float32),
        mesh=plsc.VectorSubcoreMesh(core_axis_name="c", subcore_axis_name="s",
                                    num_cores=NC, num_subcores=NS),
        scratch_types=[
            pltpu.VMEM((EPW,), jnp.int32),
            pltpu.VMEM((EPW,), jnp.int32),
            pltpu.VMEM((8, 128), jnp.int32),
            pltpu.VMEM((EPW,), jnp.float32),
            pltpu.VMEM((EPW,), jnp.float32),
            pltpu.VMEM(((nslots + 1) // 2, 8, 128), jnp.float32),
            pltpu.VMEM((ZB,), jnp.float32),
            pltpu.VMEM_SHARED((N * N,), jnp.float32),
            pltpu.SemaphoreType.DMA,
        ],
    )(functools.partial(_sc_body, lo=lo, nslots=nslots))


def _build_A(dst, src, gauss_t, lo, nslots):
    return _sc_build_fn(lo, nslots)(dst, src, gauss_t).reshape(
        nslots, 8, N, 128)


def _gauss_body(ea_ref, mu_ref, sg_ref, out_ref):
    ea = ea_ref[0]
    mu = mu_ref[...]
    sg = sg_ref[...]
    g = jnp.exp(-0.5 * (ea - mu) ** 2 / (1e-15 + sg * sg))
    rows = lax.broadcasted_iota(jnp.int32, (NGAUSS, 1), 0)
    g = jnp.where(rows == ONES_ROW, 1.0, g)
    g = jnp.where(rows == ZERO_ROW, 0.0, g)
    out_ref[...] = g[None]


def _gauss_all(ea, mu, sg, T):
    return pl.pallas_call(
        _gauss_body,
        grid=(T,),
        in_specs=[
            pl.BlockSpec((1, 1, E), lambda t: (t, 0, 0)),
            pl.BlockSpec((NGAUSS, 1), lambda t: (0, 0)),
            pl.BlockSpec((NGAUSS, 1), lambda t: (0, 0)),
        ],
        out_specs=pl.BlockSpec((1, NGAUSS, E), lambda t: (t, 0, 0)),
        out_shape=jax.ShapeDtypeStruct((T, NGAUSS, E), jnp.float32),
    )(ea.reshape(T, 1, E), mu, sg)


def _cnt_body(a_ref, out_ref):
    cnt = jnp.sum(a_ref[0, 0], axis=1, keepdims=True)
    for j in range(1, 8):
        cnt = cnt + jnp.sum(a_ref[0, j], axis=1, keepdims=True)
    out_ref[...] = 1.0 / jnp.maximum(cnt, 1.0)


def _cnt_inv(A):
    return pl.pallas_call(
        _cnt_body,
        grid=(1,),
        in_specs=[pl.BlockSpec((1, 8, N, 128), lambda k: (ONES_ROW, 0, 0, 0))],
        out_specs=pl.BlockSpec((N, 1), lambda k: (0, 0)),
        out_shape=jax.ShapeDtypeStruct((N, 1), jnp.float32),
    )(A)


def _conv_step(x, A, g3, root, bias, gamma, beta, ci, addin,
               *, cin_p, cout, K, s0, do_elu, do_pool):
    def body(x_ref, a_ref, g_ref, root_ref, b_ref, gam_ref, bet_ref, ci_ref,
             *refs):
        if addin is not None:
            addin_ref, h_ref = refs[0], refs[1]
            pool_ref = refs[2] if do_pool else None
        else:
            h_ref = refs[0]
            pool_ref = refs[1] if do_pool else None
        k = pl.program_id(0)
        xg = jnp.dot(x_ref[...], g_ref[0], preferred_element_type=jnp.float32)
        amat = jnp.concatenate([a_ref[0, j] for j in range(8)], axis=1)
        contrib = jnp.dot(amat, xg, preferred_element_type=jnp.float32)

        @pl.when(k == 0)
        def _():
            h_ref[...] = contrib

        if K > 1:
            @pl.when(k > 0)
            def _():
                h_ref[...] = h_ref[...] + contrib

        @pl.when(k == K - 1)
        def _():
            y = (h_ref[...] * ci_ref[...]
                 + jnp.dot(x_ref[...], root_ref[...],
                           preferred_element_type=jnp.float32)
                 + b_ref[...])
            m = jnp.mean(y, axis=0, keepdims=True)
            v = jnp.mean((y - m) ** 2, axis=0, keepdims=True)
            y = gam_ref[...] * (y - m) * lax.rsqrt(v + 1e-5) + bet_ref[...]
            if addin is not None:
                y = y + addin_ref[...]
            if do_elu:
                y = jnp.where(y > 0, y, jnp.exp(jnp.minimum(y, 0.0)) - 1.0)
            h_ref[...] = y
            if do_pool:
                V = N // B
                pool_ref[...] = jnp.concatenate(
                    [jnp.sum(y[:V], 0, keepdims=True),
                     jnp.sum(y[V:], 0, keepdims=True)], axis=0) * (1.0 / V)

    in_specs = [
        pl.BlockSpec((N, cin_p), lambda k: (0, 0)),
        pl.BlockSpec((1, 8, N, 128), lambda k: (s0 + k, 0, 0, 0)),
        pl.BlockSpec((1, cin_p, cout), lambda k: (k, 0, 0)),
        pl.BlockSpec((cin_p, cout), lambda k: (0, 0)),
        pl.BlockSpec((1, cout), lambda k: (0, 0)),
        pl.BlockSpec((1, cout), lambda k: (0, 0)),
        pl.BlockSpec((1, cout), lambda k: (0, 0)),
        pl.BlockSpec((N, 1), lambda k: (0, 0)),
    ]
    args = [x, A, g3, root, bias, gamma, beta, ci]
    if addin is not None:
        in_specs.append(pl.BlockSpec((N, cout), lambda k: (0, 0)))
        args.append(addin)
    out_specs = [pl.BlockSpec((N, cout), lambda k: (0, 0))]
    out_shape = [jax.ShapeDtypeStruct((N, cout), jnp.float32)]
    if do_pool:
        out_specs.append(pl.BlockSpec((B, cout), lambda k: (0, 0)))
        out_shape.append(jax.ShapeDtypeStruct((B, cout), jnp.float32))
    res = pl.pallas_call(
        body,
        grid=(K,),
        in_specs=in_specs,
        out_specs=out_specs,
        out_shape=out_shape,
    )(*args)
    return res if do_pool else (res[0], None)


def _padc(x, cin, cin_p):
    if cin_p == cin:
        return x
    return jnp.pad(x, ((0, 0), (0, cin_p - cin)))


def _prep_conv(params, idx):
    path, cin, cout, K, bnpath = _CONVS[idx]
    cin_p = max(cin, 32) if cin < 32 else cin
    p = _get(params, path)
    bn = _get(params, (bnpath,) if isinstance(bnpath, str) else bnpath)
    g3 = p["g"].reshape(cin, K, cout).transpose(1, 0, 2)
    if cin_p != cin:
        g3 = jnp.pad(g3, ((0, 0), (0, cin_p - cin), (0, 0)))
    root = p["root"]
    if cin_p != cin:
        root = jnp.pad(root, ((0, cin_p - cin), (0, 0)))
    return dict(
        g3=g3, root=root, bias=p["bias"].reshape(1, cout),
        gamma=bn["gamma"].reshape(1, cout), beta=bn["beta"].reshape(1, cout),
        cin=cin, cin_p=cin_p, cout=cout, K=K, s0=_S0[idx])


def _run_conv(cp, x, A, ci, addin=None, do_elu=True, do_pool=False, lo=0):
    return _conv_step(
        _padc(x, cp["cin"], cp["cin_p"]), A, cp["g3"], cp["root"], cp["bias"],
        cp["gamma"], cp["beta"], ci, addin,
        cin_p=cp["cin_p"], cout=cp["cout"], K=cp["K"], s0=cp["s0"] - lo,
        do_elu=do_elu, do_pool=do_pool)


def kernel(x, pos, edge_index, edge_attr, params):
    T = x.shape[0]
    ea = edge_attr[..., 0]

    mus, sgs = [], []
    for i, (path, _, _, K, _) in enumerate(_CONVS):
        if i == 2:
            mus.append(jnp.zeros((1,), jnp.float32))
            sgs.append(jnp.ones((1,), jnp.float32))
        p = _get(params, path)
        mus.append(p["mu"][:, 0])
        sgs.append(p["sigma"][:, 0])
    mus.append(jnp.zeros((1,), jnp.float32))
    sgs.append(jnp.ones((1,), jnp.float32))
    mu = jnp.concatenate(mus).reshape(NGAUSS, 1)
    sg = jnp.concatenate(sgs).reshape(NGAUSS, 1)

    gauss = _gauss_all(ea, mu, sg, T)
    cps = [_prep_conv(params, i) for i in range(len(_CONVS))]

    Aparts = [[None] * len(SPLITS) for _ in range(T)]
    for part, (lo, n) in enumerate(SPLITS):
        for t in range(T):
            dst = edge_index[t, 1]
            src = edge_index[t, 0]
            Aparts[t][part] = _build_A(dst, src, gauss[t], lo, n)

    def run_block(b, h, A, ci, lob):
        i_lc1, i_lc2, i_sc = 2 + 3 * b, 3 + 3 * b, 4 + 3 * b
        left1, _ = _run_conv(cps[i_lc1], h, A, ci, lo=lob)
        scp, _ = _run_conv(cps[i_sc], h, A, ci, do_elu=False, lo=lob)
        return _run_conv(cps[i_lc2], left1, A, ci, addin=scp,
                         do_pool=True, lo=lob)

    cis, hs, posf, pools = [], [], [], [[] for _ in range(T)]
    for t in range(T):
        A1 = Aparts[t][0]
        ci = _cnt_inv(A1)
        cis.append(ci)
        npf, pp = _run_conv(cps[0], pos[t], A1, ci, do_pool=True)
        posf.append(pp)
        h, _ = _run_conv(cps[1], x[t], A1, ci)
        hs.append(h)

    for t in range(T):
        h = hs[t]
        for b in range(2):
            h, pb = run_block(b, h, Aparts[t][1], cis[t], SPLITS[1][0])
            pools[t].append(pb)
        hs[t] = h

    feats, l1, l2, l3, l4 = [], [], [], [], []
    for t in range(T):
        h = hs[t]
        for b in range(2, 4):
            h, pb = run_block(b, h, Aparts[t][2], cis[t], SPLITS[2][0])
            pools[t].append(pb)
        l1.append(pools[t][0])
        l2.append(pools[t][1])
        l3.append(pools[t][2])
        l4.append(pools[t][3])
        feats.append(h.reshape(B, N // B, 512))

    voxel = jnp.transpose(jnp.stack(feats, axis=0), (1, 0, 2, 3))
    L1 = jnp.stack(l1, axis=1)
    L2 = jnp.stack(l2, axis=1)
    L3 = jnp.stack(l3, axis=1)
    L4 = jnp.stack(l4, axis=1)
    nodepos = jnp.transpose(jnp.stack(posf, axis=0), (1, 0, 2))
    return (voxel, L1, L2, L3, L4, nodepos)


# speedup vs baseline: 1.0116x; 1.0116x over previous
import functools

import jax
import jax.numpy as jnp
from jax import lax
from jax.experimental import pallas as pl
from jax.experimental.pallas import tpu as pltpu
from jax.experimental.pallas import tpu_sc as plsc

N = 1024
E = 16384
B = 2
NC, NS = 2, 16
EPW = E // NS
WPW = (N * N) // NS
ZB = 4096

_CONVS = [
    (("conv_node",), 3, 32, 5, "bn_node"),
    (("conv_0",), 30, 32, 5, "bn_0"),
    (("block1", "lc1"), 32, 64, 5, ("block1", "lb1")),
    (("block1", "lc2"), 64, 64, 5, ("block1", "lb2")),
    (("block1", "sc"), 32, 64, 1, ("block1", "sb")),
    (("block2", "lc1"), 64, 128, 5, ("block2", "lb1")),
    (("block2", "lc2"), 128, 128, 5, ("block2", "lb2")),
    (("block2", "sc"), 64, 128, 1, ("block2", "sb")),
    (("block3", "lc1"), 128, 256, 5, ("block3", "lb1")),
    (("block3", "lc2"), 256, 256, 5, ("block3", "lb2")),
    (("block3", "sc"), 128, 256, 1, ("block3", "sb")),
    (("block4", "lc1"), 256, 512, 5, ("block4", "lb1")),
    (("block4", "lc2"), 512, 512, 5, ("block4", "lb2")),
    (("block4", "sc"), 256, 512, 1, ("block4", "sb")),
]
_S0 = []
_acc = 0
for _i, _c in enumerate(_CONVS):
    if _i == 8:
        _acc += 1
    _S0.append(_acc)
    _acc += _c[3]
ONES_ROW = 32
NSLOT = _acc
SPLITS = ((0, 33), (33, 22))
_PART_FOR_BLOCK = (0, 0, 1, 1)
ZERO_ROW = NSLOT
NGAUSS = NSLOT + 1


def _get(params, path):
    v = params
    for k in path:
        v = v[k]
    return v


def _sc_body(dst_hbm, src_hbm, gauss_hbm, a_out,
             dv, sv, idx2d, gnew, gold, delta3, zbuf, a_sp, sem,
             *, lo, nslots):
    c = lax.axis_index("c")
    s = lax.axis_index("s")
    base_e = s * EPW

    pltpu.sync_copy(dst_hbm.at[pl.ds(base_e, EPW)], dv)
    pltpu.sync_copy(src_hbm.at[pl.ds(base_e, EPW)], sv)

    for j in range(8):
        def mk(i, _, j=j):
            d = dv[pl.ds(j * 128 + i * 16, 16)]
            r = sv[pl.ds(j * 128 + i * 16, 16)]
            idx2d[j, pl.ds(i * 16, 16)] = (
                lax.shift_left(lax.shift_right_logical(r, 7), 17)
                + lax.shift_left(d, 7) + lax.bitwise_and(r, 127))
            return 0
        lax.fori_loop(0, 8, mk, 0)

    def z(i, _):
        zbuf[pl.ds(i * 16, 16)] = jnp.zeros((16,), jnp.float32)
        return 0
    lax.fori_loop(0, ZB // 16, z, 0)
    for q in range(WPW // ZB):
        pltpu.sync_copy(zbuf, a_sp.at[pl.ds(s * WPW + q * ZB, ZB)])

    n_iter = (nslots + 1) // 2
    hi = lo + nslots

    def mkdelta(i, _):
        slot = lo + 2 * i + c
        prev = jnp.where(i == 0, ZERO_ROW, slot - 2)
        sl = jnp.where(slot < hi, slot, ZERO_ROW)
        c1 = pltpu.async_copy(gauss_hbm.at[sl, pl.ds(base_e, EPW)], gnew, sem)
        c2 = pltpu.async_copy(gauss_hbm.at[prev, pl.ds(base_e, EPW)], gold,
                              sem)
        c1.wait()
        c2.wait()
        for j in range(8):
            def db(i2, _, j=j):
                delta3[i, j, pl.ds(i2 * 16, 16)] = (
                    gnew[pl.ds(j * 128 + i2 * 16, 16)]
                    - gold[pl.ds(j * 128 + i2 * 16, 16)])
                return 0
            lax.fori_loop(0, 8, db, 0)
        return 0

    lax.fori_loop(0, n_iter, mkdelta, 0)
    plsc.subcore_barrier()

    def slot_body(i, _):
        slot = lo + 2 * i + c

        @pl.when(slot < hi)
        def _():
            cps = [pltpu.async_copy(delta3.at[i, j], a_sp.at[idx2d.at[j]],
                                    sem, add=True) for j in range(8)]
            for cp in cps:
                cp.wait()
        plsc.subcore_barrier()

        @pl.when(slot < hi)
        def _():
            pltpu.sync_copy(
                a_sp.at[pl.ds(s * WPW, WPW)],
                a_out.at[pl.ds((slot - lo) * (N * N) + s * WPW, WPW)])
        plsc.subcore_barrier()
        return 0

    lax.fori_loop(0, n_iter, slot_body, 0)


@functools.cache
def _sc_build_fn(lo, nslots):
    return functools.partial(
        pl.kernel,
        out_type=jax.ShapeDtypeStruct((nslots * N * N,), jnp.float32),
        mesh=plsc.VectorSubcoreMesh(core_axis_name="c", subcore_axis_name="s",
                                    num_cores=NC, num_subcores=NS),
        scratch_types=[
            pltpu.VMEM((EPW,), jnp.int32),
            pltpu.VMEM((EPW,), jnp.int32),
            pltpu.VMEM((8, 128), jnp.int32),
            pltpu.VMEM((EPW,), jnp.float32),
            pltpu.VMEM((EPW,), jnp.float32),
            pltpu.VMEM(((nslots + 1) // 2, 8, 128), jnp.float32),
            pltpu.VMEM((ZB,), jnp.float32),
            pltpu.VMEM_SHARED((N * N,), jnp.float32),
            pltpu.SemaphoreType.DMA,
        ],
    )(functools.partial(_sc_body, lo=lo, nslots=nslots))


def _build_A(dst, src, gauss_t, lo, nslots):
    return _sc_build_fn(lo, nslots)(dst, src, gauss_t).reshape(
        nslots, 8, N, 128)


def _gauss_body(ea_ref, mu_ref, sg_ref, out_ref):
    ea = ea_ref[0]
    mu = mu_ref[...]
    sg = sg_ref[...]
    g = jnp.exp(-0.5 * (ea - mu) ** 2 / (1e-15 + sg * sg))
    rows = lax.broadcasted_iota(jnp.int32, (NGAUSS, 1), 0)
    g = jnp.where(rows == ONES_ROW, 1.0, g)
    g = jnp.where(rows == ZERO_ROW, 0.0, g)
    out_ref[...] = g[None]


def _gauss_all(ea, mu, sg, T):
    return pl.pallas_call(
        _gauss_body,
        grid=(T,),
        in_specs=[
            pl.BlockSpec((1, 1, E), lambda t: (t, 0, 0)),
            pl.BlockSpec((NGAUSS, 1), lambda t: (0, 0)),
            pl.BlockSpec((NGAUSS, 1), lambda t: (0, 0)),
        ],
        out_specs=pl.BlockSpec((1, NGAUSS, E), lambda t: (t, 0, 0)),
        out_shape=jax.ShapeDtypeStruct((T, NGAUSS, E), jnp.float32),
    )(ea.reshape(T, 1, E), mu, sg)


def _cnt_body(a_ref, out_ref):
    cnt = jnp.sum(a_ref[0, 0], axis=1, keepdims=True)
    for j in range(1, 8):
        cnt = cnt + jnp.sum(a_ref[0, j], axis=1, keepdims=True)
    out_ref[...] = 1.0 / jnp.maximum(cnt, 1.0)


def _cnt_inv(A):
    return pl.pallas_call(
        _cnt_body,
        grid=(1,),
        in_specs=[pl.BlockSpec((1, 8, N, 128), lambda k: (ONES_ROW, 0, 0, 0))],
        out_specs=pl.BlockSpec((N, 1), lambda k: (0, 0)),
        out_shape=jax.ShapeDtypeStruct((N, 1), jnp.float32),
    )(A)


def _conv_step(x, A, g3, root, bias, gamma, beta, ci, addin,
               *, cin_p, cout, K, s0, do_elu, do_pool):
    def body(x_ref, a_ref, g_ref, root_ref, b_ref, gam_ref, bet_ref, ci_ref,
             *refs):
        if addin is not None:
            addin_ref, h_ref = refs[0], refs[1]
            pool_ref = refs[2] if do_pool else None
        else:
            h_ref = refs[0]
            pool_ref = refs[1] if do_pool else None
        k = pl.program_id(0)
        xg = jnp.dot(x_ref[...], g_ref[0], preferred_element_type=jnp.float32)
        amat = jnp.concatenate([a_ref[0, j] for j in range(8)], axis=1)
        contrib = jnp.dot(amat, xg, preferred_element_type=jnp.float32)

        @pl.when(k == 0)
        def _():
            h_ref[...] = contrib

        if K > 1:
            @pl.when(k > 0)
            def _():
                h_ref[...] = h_ref[...] + contrib

        @pl.when(k == K - 1)
        def _():
            y = (h_ref[...] * ci_ref[...]
                 + jnp.dot(x_ref[...], root_ref[...],
                           preferred_element_type=jnp.float32)
                 + b_ref[...])
            m = jnp.mean(y, axis=0, keepdims=True)
            v = jnp.mean((y - m) ** 2, axis=0, keepdims=True)
            y = gam_ref[...] * (y - m) * lax.rsqrt(v + 1e-5) + bet_ref[...]
            if addin is not None:
                y = y + addin_ref[...]
            if do_elu:
                y = jnp.where(y > 0, y, jnp.exp(jnp.minimum(y, 0.0)) - 1.0)
            h_ref[...] = y
            if do_pool:
                V = N // B
                pool_ref[...] = jnp.concatenate(
                    [jnp.sum(y[:V], 0, keepdims=True),
                     jnp.sum(y[V:], 0, keepdims=True)], axis=0) * (1.0 / V)

    in_specs = [
        pl.BlockSpec((N, cin_p), lambda k: (0, 0)),
        pl.BlockSpec((1, 8, N, 128), lambda k: (s0 + k, 0, 0, 0)),
        pl.BlockSpec((1, cin_p, cout), lambda k: (k, 0, 0)),
        pl.BlockSpec((cin_p, cout), lambda k: (0, 0)),
        pl.BlockSpec((1, cout), lambda k: (0, 0)),
        pl.BlockSpec((1, cout), lambda k: (0, 0)),
        pl.BlockSpec((1, cout), lambda k: (0, 0)),
        pl.BlockSpec((N, 1), lambda k: (0, 0)),
    ]
    args = [x, A, g3, root, bias, gamma, beta, ci]
    if addin is not None:
        in_specs.append(pl.BlockSpec((N, cout), lambda k: (0, 0)))
        args.append(addin)
    out_specs = [pl.BlockSpec((N, cout), lambda k: (0, 0))]
    out_shape = [jax.ShapeDtypeStruct((N, cout), jnp.float32)]
    if do_pool:
        out_specs.append(pl.BlockSpec((B, cout), lambda k: (0, 0)))
        out_shape.append(jax.ShapeDtypeStruct((B, cout), jnp.float32))
    res = pl.pallas_call(
        body,
        grid=(K,),
        in_specs=in_specs,
        out_specs=out_specs,
        out_shape=out_shape,
    )(*args)
    return res if do_pool else (res[0], None)


def _padc(x, cin, cin_p):
    if cin_p == cin:
        return x
    return jnp.pad(x, ((0, 0), (0, cin_p - cin)))


def _prep_conv(params, idx):
    path, cin, cout, K, bnpath = _CONVS[idx]
    cin_p = max(cin, 32) if cin < 32 else cin
    p = _get(params, path)
    bn = _get(params, (bnpath,) if isinstance(bnpath, str) else bnpath)
    g3 = p["g"].reshape(cin, K, cout).transpose(1, 0, 2)
    if cin_p != cin:
        g3 = jnp.pad(g3, ((0, 0), (0, cin_p - cin), (0, 0)))
    root = p["root"]
    if cin_p != cin:
        root = jnp.pad(root, ((0, cin_p - cin), (0, 0)))
    return dict(
        g3=g3, root=root, bias=p["bias"].reshape(1, cout),
        gamma=bn["gamma"].reshape(1, cout), beta=bn["beta"].reshape(1, cout),
        cin=cin, cin_p=cin_p, cout=cout, K=K, s0=_S0[idx])


def _run_conv(cp, x, A, ci, addin=None, do_elu=True, do_pool=False, lo=0):
    return _conv_step(
        _padc(x, cp["cin"], cp["cin_p"]), A, cp["g3"], cp["root"], cp["bias"],
        cp["gamma"], cp["beta"], ci, addin,
        cin_p=cp["cin_p"], cout=cp["cout"], K=cp["K"], s0=cp["s0"] - lo,
        do_elu=do_elu, do_pool=do_pool)


def kernel(x, pos, edge_index, edge_attr, params):
    T = x.shape[0]
    ea = edge_attr[..., 0]

    mus, sgs = [], []
    for i, (path, _, _, K, _) in enumerate(_CONVS):
        if i == 8:
            mus.append(jnp.zeros((1,), jnp.float32))
            sgs.append(jnp.ones((1,), jnp.float32))
        p = _get(params, path)
        mus.append(p["mu"][:, 0])
        sgs.append(p["sigma"][:, 0])
    mus.append(jnp.zeros((1,), jnp.float32))
    sgs.append(jnp.ones((1,), jnp.float32))
    mu = jnp.concatenate(mus).reshape(NGAUSS, 1)
    sg = jnp.concatenate(sgs).reshape(NGAUSS, 1)

    gauss = _gauss_all(ea, mu, sg, T)
    cps = [_prep_conv(params, i) for i in range(len(_CONVS))]

    Aparts = [[None] * len(SPLITS) for _ in range(T)]
    for part, (lo, n) in enumerate(SPLITS):
        for t in range(T):
            dst = edge_index[t, 1]
            src = edge_index[t, 0]
            Aparts[t][part] = _build_A(dst, src, gauss[t], lo, n)

    def run_block(b, h, A, ci, lob):
        i_lc1, i_lc2, i_sc = 2 + 3 * b, 3 + 3 * b, 4 + 3 * b
        left1, _ = _run_conv(cps[i_lc1], h, A, ci, lo=lob)
        scp, _ = _run_conv(cps[i_sc], h, A, ci, do_elu=False, lo=lob)
        return _run_conv(cps[i_lc2], left1, A, ci, addin=scp,
                         do_pool=True, lo=lob)

    cis, hs, posf, pools = [], [], [], [[] for _ in range(T)]
    for t in range(T):
        A1 = Aparts[t][0]
        ci = _cnt_inv(A1)
        cis.append(ci)
        npf, pp = _run_conv(cps[0], pos[t], A1, ci, do_pool=True)
        posf.append(pp)
        h, _ = _run_conv(cps[1], x[t], A1, ci)
        hs.append(h)

    for t in range(T):
        h = hs[t]
        for b in range(2):
            p = _PART_FOR_BLOCK[b]
            h, pb = run_block(b, h, Aparts[t][p], cis[t], SPLITS[p][0])
            pools[t].append(pb)
        hs[t] = h

    feats, l1, l2, l3, l4 = [], [], [], [], []
    for t in range(T):
        h = hs[t]
        for b in range(2, 4):
            p = _PART_FOR_BLOCK[b]
            h, pb = run_block(b, h, Aparts[t][p], cis[t], SPLITS[p][0])
            pools[t].append(pb)
        l1.append(pools[t][0])
        l2.append(pools[t][1])
        l3.append(pools[t][2])
        l4.append(pools[t][3])
        feats.append(h.reshape(B, N // B, 512))

    voxel = jnp.transpose(jnp.stack(feats, axis=0), (1, 0, 2, 3))
    L1 = jnp.stack(l1, axis=1)
    L2 = jnp.stack(l2, axis=1)
    L3 = jnp.stack(l3, axis=1)
    L4 = jnp.stack(l4, axis=1)
    nodepos = jnp.transpose(jnp.stack(posf, axis=0), (1, 0, 2))
    return (voxel, L1, L2, L3, L4, nodepos)
